# initial kernel scaffold (unmeasured)
import numpy as np
import jax
import jax.numpy as jnp
from jax import lax
from jax.experimental import pallas as pl
from jax.experimental.pallas import tpu as pltpu

N_DEV = 4
SEQ = 1024
D = 1024
HQ = 8
DH = 128
SCALE = 0.08838834764831843

_P = np.zeros((DH, DH), dtype=np.float32)
for _i in range(DH // 2):
    _P[2 * _i + 1, 2 * _i] = -1.0
    _P[2 * _i, 2 * _i + 1] = 1.0

_INV = (1.0 / (10000.0 ** (np.arange(0, DH, 2) / DH))).astype(np.float32)


def kernel(x, Wq, Wk, Wv, Wo):
    xs = x.reshape(SEQ, D)

    my = lax.axis_index("i")
    pos = my * SEQ + jnp.arange(SEQ, dtype=jnp.float32)
    ang = pos[:, None] * jnp.asarray(_INV)[None, :]
    cos = jnp.repeat(jnp.cos(ang), 2, axis=-1)
    sin = jnp.repeat(jnp.sin(ang), 2, axis=-1)
    P = jnp.asarray(_P)

    def body(x_ref, wq_ref, wk_ref, wv_ref, wo_ref, cos_ref, sin_ref, p_ref,
             out_ref, kbuf, vbuf, ksend, krecv, vsend, vrecv):
        my_pos = lax.axis_index("i")
        left = lax.rem(my_pos + N_DEV - 1, N_DEV)
        right = lax.rem(my_pos + 1, N_DEV)

        cosv = cos_ref[...]
        sinv = sin_ref[...]
        pm = p_ref[...]

        def rope(t):
            return t * cosv + jnp.dot(
                t, pm, preferred_element_type=jnp.float32) * sinv

        xv = x_ref[...]

        k = jnp.dot(xv, wk_ref[...], preferred_element_type=jnp.float32)
        kr = jnp.concatenate(
            [rope(k[:, h * DH:(h + 1) * DH]) for h in range(HQ)], axis=-1)
        v = jnp.dot(xv, wv_ref[...], preferred_element_type=jnp.float32)
        kbuf[pl.ds(my_pos, 1)] = kr[None]
        vbuf[pl.ds(my_pos, 1)] = v[None]

        barrier_sem = pltpu.get_barrier_semaphore()
        pl.semaphore_signal(barrier_sem, inc=1, device_id=(left,),
                            device_id_type=pl.DeviceIdType.MESH)
        pl.semaphore_signal(barrier_sem, inc=1, device_id=(right,),
                            device_id_type=pl.DeviceIdType.MESH)
        pl.semaphore_wait(barrier_sem, 2)

        for h in range(N_DEV - 1):
            slot = lax.rem(my_pos + N_DEV - h, N_DEV)
            k_rdma = pltpu.make_async_remote_copy(
                src_ref=kbuf.at[slot], dst_ref=kbuf.at[slot],
                send_sem=ksend.at[h], recv_sem=krecv.at[h],
                device_id=(right,), device_id_type=pl.DeviceIdType.MESH)
            v_rdma = pltpu.make_async_remote_copy(
                src_ref=vbuf.at[slot], dst_ref=vbuf.at[slot],
                send_sem=vsend.at[h], recv_sem=vrecv.at[h],
                device_id=(right,), device_id_type=pl.DeviceIdType.MESH)
            k_rdma.start()
            v_rdma.start()
            k_rdma.wait()
            v_rdma.wait()

        q = jnp.dot(xv, wq_ref[...], preferred_element_type=jnp.float32)
        ctx_heads = []
        for hh in range(HQ):
            sl = slice(hh * DH, (hh + 1) * DH)
            qh = rope(q[:, sl])
            kh = jnp.concatenate(
                [kbuf[j, :, sl] for j in range(N_DEV)], axis=0)
            vh = jnp.concatenate(
                [vbuf[j, :, sl] for j in range(N_DEV)], axis=0)
            s = lax.dot_general(
                qh, kh, (((1,), (1,)), ((), ())),
                preferred_element_type=jnp.float32) * SCALE
            m = jnp.max(s, axis=-1, keepdims=True)
            e = jnp.exp(s - m)
            w = e / jnp.sum(e, axis=-1, keepdims=True)
            ctx_heads.append(
                jnp.dot(w, vh, preferred_element_type=jnp.float32))
        ctx = jnp.concatenate(ctx_heads, axis=-1)
        out_ref[...] = jnp.dot(
            ctx, wo_ref[...], preferred_element_type=jnp.float32)

    out2d = pl.pallas_call(
        body,
        out_shape=jax.ShapeDtypeStruct((SEQ, D), jnp.float32),
        in_specs=[pl.BlockSpec(memory_space=pltpu.VMEM)] * 8,
        out_specs=pl.BlockSpec(memory_space=pltpu.VMEM),
        scratch_shapes=[
            pltpu.VMEM((N_DEV, SEQ, D), jnp.float32),
            pltpu.VMEM((N_DEV, SEQ, D), jnp.float32),
            pltpu.SemaphoreType.DMA((N_DEV - 1,)),
            pltpu.SemaphoreType.DMA((N_DEV - 1,)),
            pltpu.SemaphoreType.DMA((N_DEV - 1,)),
            pltpu.SemaphoreType.DMA((N_DEV - 1,)),
        ],
        compiler_params=pltpu.CompilerParams(collective_id=0),
    )(xs, Wq, Wk, Wv, Wo, cos, sin, P)

    return out2d.reshape(1, SEQ, D)


# baseline (device time: 358719 ns/iter reference)
import numpy as np
import jax
import jax.numpy as jnp
from jax import lax
from jax.experimental import pallas as pl
from jax.experimental.pallas import tpu as pltpu

N_DEV = 4
SEQ = 1024
D = 1024
HQ = 8
DH = 128
QT = 512
SCALE = 0.08838834764831843

_INV = (1.0 / (10000.0 ** (np.arange(0, DH, 2) / DH))).astype(np.float32)


def kernel(x, Wq, Wk, Wv, Wo):
    xs = x.reshape(SEQ, D)

    my = lax.axis_index("i")
    pos = my * SEQ + jnp.arange(SEQ, dtype=jnp.float32)
    ang = pos[:, None] * jnp.asarray(_INV)[None, :]
    cos = jnp.repeat(jnp.cos(ang), 2, axis=-1)
    sin = jnp.repeat(jnp.sin(ang), 2, axis=-1)

    def rope(t):
        t2 = t.reshape(SEQ, HQ, DH // 2, 2)
        tr = jnp.stack([-t2[..., 1], t2[..., 0]], axis=-1).reshape(SEQ, HQ, DH)
        return t * cos[:, None, :] + tr * sin[:, None, :]

    q = (rope((xs @ Wq).reshape(SEQ, HQ, DH)) * SCALE).transpose(1, 0, 2)
    k = rope((xs @ Wk).reshape(SEQ, HQ, DH)).transpose(1, 0, 2)
    v = (xs @ Wv).reshape(SEQ, HQ, DH).transpose(1, 0, 2)
    wo = Wo.reshape(HQ, DH, D)

    def body(q_ref, k_ref, v_ref, wo_ref, out_ref,
             kcom, vcom, acc_ref, den_ref, ksend, krecv, vsend, vrecv,
             credit):
        my_pos = lax.axis_index("i")
        left = lax.rem(my_pos + N_DEV - 1, N_DEV)
        right = lax.rem(my_pos + 1, N_DEV)

        barrier_sem = pltpu.get_barrier_semaphore()
        pl.semaphore_signal(barrier_sem, inc=1, device_id=(left,),
                            device_id_type=pl.DeviceIdType.MESH)
        pl.semaphore_signal(barrier_sem, inc=1, device_id=(right,),
                            device_id_type=pl.DeviceIdType.MESH)
        pl.semaphore_wait(barrier_sem, 2)

        for t in range(N_DEV):
            if t < N_DEV - 1:
                if t == 2:
                    pl.semaphore_wait(credit, 1)
                ksrc = k_ref if t == 0 else kcom.at[(t + 1) % 2]
                vsrc = v_ref if t == 0 else vcom.at[(t + 1) % 2]
                k_rdma = pltpu.make_async_remote_copy(
                    src_ref=ksrc, dst_ref=kcom.at[t % 2],
                    send_sem=ksend.at[t], recv_sem=krecv.at[t],
                    device_id=(right,), device_id_type=pl.DeviceIdType.MESH)
                v_rdma = pltpu.make_async_remote_copy(
                    src_ref=vsrc, dst_ref=vcom.at[t % 2],
                    send_sem=vsend.at[t], recv_sem=vrecv.at[t],
                    device_id=(right,), device_id_type=pl.DeviceIdType.MESH)
                k_rdma.start()
                v_rdma.start()

            slot = (t + 1) % 2

            def head_step(h, _, t=t, slot=slot):
                kc = k_ref[h] if t == 0 else kcom[slot, h]
                vc = v_ref[h] if t == 0 else vcom[slot, h]
                for c in range(SEQ // QT):
                    rows = pl.ds(c * QT, QT)
                    e = jnp.exp(lax.dot_general(
                        q_ref[h, rows], kc, (((1,), (1,)), ((), ())),
                        preferred_element_type=jnp.float32))
                    num = jnp.dot(e, vc, preferred_element_type=jnp.float32)
                    den = jnp.sum(e, axis=-1, keepdims=True)
                    if t == 0:
                        acc_ref[h, rows] = num
                        den_ref[h, rows] = den
                    else:
                        acc_ref[h, rows] = acc_ref[h, rows] + num
                        den_ref[h, rows] = den_ref[h, rows] + den
                return _

            lax.fori_loop(0, HQ, head_step, 0)

            if t < N_DEV - 1:
                k_rdma.wait()
                v_rdma.wait()
            if t == 1:
                pl.semaphore_signal(credit, inc=1, device_id=(left,),
                                    device_id_type=pl.DeviceIdType.MESH)

        out_ref[...] = jnp.zeros((SEQ, D), jnp.float32)

        def proj_step(h, _):
            ctx_h = acc_ref[h] / den_ref[h]
            out_ref[...] = out_ref[...] + jnp.dot(
                ctx_h, wo_ref[h], preferred_element_type=jnp.float32)
            return _

        lax.fori_loop(0, HQ, proj_step, 0)

    out2d = pl.pallas_call(
        body,
        out_shape=jax.ShapeDtypeStruct((SEQ, D), jnp.float32),
        in_specs=[pl.BlockSpec(memory_space=pltpu.VMEM)] * 4,
        out_specs=pl.BlockSpec(memory_space=pltpu.VMEM),
        scratch_shapes=[
            pltpu.VMEM((2, HQ, SEQ, DH), jnp.float32),
            pltpu.VMEM((2, HQ, SEQ, DH), jnp.float32),
            pltpu.VMEM((HQ, SEQ, DH), jnp.float32),
            pltpu.VMEM((HQ, SEQ, 1), jnp.float32),
            pltpu.SemaphoreType.DMA((N_DEV - 1,)),
            pltpu.SemaphoreType.DMA((N_DEV - 1,)),
            pltpu.SemaphoreType.DMA((N_DEV - 1,)),
            pltpu.SemaphoreType.DMA((N_DEV - 1,)),
            pltpu.SemaphoreType.REGULAR,
        ],
        compiler_params=pltpu.CompilerParams(
            collective_id=0, vmem_limit_bytes=100 * 1024 * 1024),
    )(q, k, v, wo)

    return out2d.reshape(1, SEQ, D)


# device time: 222093 ns/iter; 1.6152x vs baseline; 1.6152x over previous
import numpy as np
import jax
import jax.numpy as jnp
from jax import lax
from jax.experimental import pallas as pl
from jax.experimental.pallas import tpu as pltpu

N_DEV = 4
SEQ = 1024
D = 1024
HQ = 8
DH = 128
QT = 512
HALF = SEQ // 2
SCALE = 0.08838834764831843

_INV = (1.0 / (10000.0 ** (np.arange(0, DH, 2) / DH))).astype(np.float32)


def kernel(x, Wq, Wk, Wv, Wo):
    xs = x.reshape(SEQ, D)

    my = lax.axis_index("i")
    pos = my * SEQ + jnp.arange(SEQ, dtype=jnp.float32)
    ang = pos[:, None] * jnp.asarray(_INV)[None, :]
    cos = jnp.repeat(jnp.cos(ang), 2, axis=-1)
    sin = jnp.repeat(jnp.sin(ang), 2, axis=-1)

    def rope(t):
        t2 = t.reshape(SEQ, HQ, DH // 2, 2)
        tr = jnp.stack([-t2[..., 1], t2[..., 0]], axis=-1).reshape(SEQ, HQ, DH)
        return t * cos[:, None, :] + tr * sin[:, None, :]

    q = (rope((xs @ Wq).reshape(SEQ, HQ, DH)) * SCALE).transpose(1, 0, 2)
    k = rope((xs @ Wk).reshape(SEQ, HQ, DH)).transpose(1, 0, 2)
    v = (xs @ Wv).reshape(SEQ, HQ, DH).transpose(1, 0, 2)
    kv = jnp.stack([k, v])
    wo = Wo.reshape(HQ, DH, D)

    def body(q_ref, kv_ref, wo_ref, out_ref,
             kvcom, acc_ref, den_ref, send, recv):
        my_pos = lax.axis_index("i")
        left = lax.rem(my_pos + N_DEV - 1, N_DEV)
        right = lax.rem(my_pos + 1, N_DEV)

        barrier_sem = pltpu.get_barrier_semaphore()
        pl.semaphore_signal(barrier_sem, inc=1, device_id=(left,),
                            device_id_type=pl.DeviceIdType.MESH)
        pl.semaphore_signal(barrier_sem, inc=1, device_id=(right,),
                            device_id_type=pl.DeviceIdType.MESH)
        pl.semaphore_wait(barrier_sem, 2)

        def accumulate(src, t):
            def head_step(h, carry):
                kc = src[0, h]
                vc = src[1, h]
                for c in range(SEQ // QT):
                    rows = pl.ds(c * QT, QT)
                    e = jnp.exp(lax.dot_general(
                        q_ref[h, rows], kc, (((1,), (1,)), ((), ())),
                        preferred_element_type=jnp.float32))
                    num = jnp.dot(e, vc, preferred_element_type=jnp.float32)
                    den = jnp.sum(e, axis=-1, keepdims=True)
                    if t == 0:
                        acc_ref[h, rows] = num
                        den_ref[h, rows] = den
                    else:
                        acc_ref[h, rows] = acc_ref[h, rows] + num
                        den_ref[h, rows] = den_ref[h, rows] + den
                return carry

            lax.fori_loop(0, HQ, head_step, 0)

        p1r = pltpu.make_async_remote_copy(
            src_ref=kv_ref, dst_ref=kvcom.at[0],
            send_sem=send.at[0], recv_sem=recv.at[0],
            device_id=(right,), device_id_type=pl.DeviceIdType.MESH)
        p1l = pltpu.make_async_remote_copy(
            src_ref=kv_ref, dst_ref=kvcom.at[1],
            send_sem=send.at[1], recv_sem=recv.at[1],
            device_id=(left,), device_id_type=pl.DeviceIdType.MESH)
        p1r.start()
        p1l.start()

        accumulate(kv_ref, 0)

        p1r.wait()
        p1l.wait()

        p2r = pltpu.make_async_remote_copy(
            src_ref=kvcom.at[0, :, :, pl.ds(0, HALF)],
            dst_ref=kvcom.at[2, :, :, pl.ds(0, HALF)],
            send_sem=send.at[2], recv_sem=recv.at[2],
            device_id=(right,), device_id_type=pl.DeviceIdType.MESH)
        p2l = pltpu.make_async_remote_copy(
            src_ref=kvcom.at[1, :, :, pl.ds(HALF, HALF)],
            dst_ref=kvcom.at[2, :, :, pl.ds(HALF, HALF)],
            send_sem=send.at[3], recv_sem=recv.at[3],
            device_id=(left,), device_id_type=pl.DeviceIdType.MESH)
        p2r.start()
        p2l.start()

        accumulate(kvcom.at[0], 1)
        accumulate(kvcom.at[1], 2)

        p2r.wait()
        p2l.wait()

        accumulate(kvcom.at[2], 3)

        out_ref[...] = jnp.zeros((SEQ, D), jnp.float32)

        def proj_step(h, carry):
            ctx_h = acc_ref[h] / den_ref[h]
            out_ref[...] = out_ref[...] + jnp.dot(
                ctx_h, wo_ref[h], preferred_element_type=jnp.float32)
            return carry

        lax.fori_loop(0, HQ, proj_step, 0)

    out2d = pl.pallas_call(
        body,
        out_shape=jax.ShapeDtypeStruct((SEQ, D), jnp.float32),
        in_specs=[pl.BlockSpec(memory_space=pltpu.VMEM)] * 3,
        out_specs=pl.BlockSpec(memory_space=pltpu.VMEM),
        scratch_shapes=[
            pltpu.VMEM((3, 2, HQ, SEQ, DH), jnp.float32),
            pltpu.VMEM((HQ, SEQ, DH), jnp.float32),
            pltpu.VMEM((HQ, SEQ, 1), jnp.float32),
            pltpu.SemaphoreType.DMA((4,)),
            pltpu.SemaphoreType.DMA((4,)),
        ],
        compiler_params=pltpu.CompilerParams(
            collective_id=0, vmem_limit_bytes=100 * 1024 * 1024),
    )(q, kv, wo)

    return out2d.reshape(1, SEQ, D)
